# Initial kernel scaffold; baseline (speedup 1.0000x reference)
import jax
import jax.numpy as jnp
from jax.experimental import pallas as pl


def kernel(x, edge_index, batch, enc_W, enc_b, msg_W1, msg_b1, msg_W2, msg_b2, gru_Wih, gru_bih, gru_Whh, gru_bhh, head_W1, head_b1, head_W2, head_b2):
    raise NotImplementedError("write your pallas kernel here")


# speedup vs baseline: 6.3502x; 6.3502x over previous
import functools

import jax
import jax.numpy as jnp
from jax import lax
from jax.experimental import pallas as pl
from jax.experimental.pallas import tpu as pltpu
from jax.experimental.pallas import tpu_sc as plsc

N = 10000
E = 320000
IN = 128
H = 128
L = 6
G = 64

NW = 32
K = 128
NCH = 79
E_PAD = NW * NCH * K
N_ACC = 10008
RPT = 624

R = 1000


def _dotT(a, w):
    return lax.dot_general(a, w, (((1,), (1,)), ((), ())),
                           preferred_element_type=jnp.float32)


def _enc_msg_body(x_ref, encW_ref, encb_ref, W1_ref, b1_ref, W2_ref, b2_ref,
                  h_ref, P_ref):
    h = _dotT(x_ref[...], encW_ref[...]) + encb_ref[...]
    h_ref[...] = h
    hid = jnp.maximum(_dotT(h, W1_ref[...]) + b1_ref[...], 0.0)
    P_ref[...] = _dotT(hid, W2_ref[...]) + b2_ref[...]


def _gru_core(m0_ref, m1_ref, P_ref, h_ref, Wih_ref, bih_ref, Whh_ref, bhh_ref):
    m = m0_ref[...] + m1_ref[...] - P_ref[...]
    h = h_ref[...]
    gi = _dotT(m, Wih_ref[...]) + bih_ref[...]
    gh = _dotT(h, Whh_ref[...]) + bhh_ref[...]
    r = jax.nn.sigmoid(gi[:, :H] + gh[:, :H])
    z = jax.nn.sigmoid(gi[:, H:2 * H] + gh[:, H:2 * H])
    n = jnp.tanh(gi[:, 2 * H:] + r * gh[:, 2 * H:])
    return (1.0 - z) * n + z * h


def _gru_msg_body(m0_ref, m1_ref, P_ref, h_ref, Wih_ref, bih_ref, Whh_ref,
                  bhh_ref, W1_ref, b1_ref, W2_ref, b2_ref, hout_ref, Pout_ref):
    hn = _gru_core(m0_ref, m1_ref, P_ref, h_ref, Wih_ref, bih_ref, Whh_ref,
                   bhh_ref)
    hout_ref[...] = hn
    hid = jnp.maximum(_dotT(hn, W1_ref[...]) + b1_ref[...], 0.0)
    Pout_ref[...] = _dotT(hid, W2_ref[...]) + b2_ref[...]


def _gru_last_body(m0_ref, m1_ref, P_ref, h_ref, Wih_ref, bih_ref, Whh_ref,
                   bhh_ref, hout_ref):
    hout_ref[...] = _gru_core(m0_ref, m1_ref, P_ref, h_ref, Wih_ref, bih_ref,
                              Whh_ref, bhh_ref)


def _pool_head_body(h_ref, batch_ref, W1_ref, b1_ref, W2_ref, b2_ref, out_ref):
    h = h_ref[...]
    b = batch_ref[...]
    gids = lax.broadcasted_iota(jnp.int32, (1, G), 1)
    oh = (b == gids).astype(jnp.float32)
    sums = lax.dot_general(oh, h, (((0,), (0,)), ((), ())),
                           preferred_element_type=jnp.float32)
    counts = lax.dot_general(oh, jnp.ones((N, 1), jnp.float32),
                             (((0,), (0,)), ((), ())),
                             preferred_element_type=jnp.float32)
    pooled = sums / jnp.maximum(counts, 1.0)
    ph = jnp.maximum(_dotT(pooled, W1_ref[...]) + b1_ref[...], 0.0)
    out_ref[...] = jnp.sum(ph * W2_ref[...], axis=1, keepdims=True) + b2_ref[0, 0]


def _row_spec(shape):
    return pl.BlockSpec(shape, lambda i: (i, 0))


def _full_spec(shape):
    return pl.BlockSpec(shape, lambda i: (0, 0))


def _enc_msg(x, encW, encb, W1, b1, W2, b2):
    return pl.pallas_call(
        _enc_msg_body,
        grid=(N // R,),
        in_specs=[
            _row_spec((R, IN)),
            _full_spec((H, IN)), _full_spec((1, H)),
            _full_spec((H, H)), _full_spec((1, H)),
            _full_spec((H, H)), _full_spec((1, H)),
        ],
        out_specs=[_row_spec((R, H)), _row_spec((R, H))],
        out_shape=[jax.ShapeDtypeStruct((N, H), jnp.float32)] * 2,
    )(x, encW, encb, W1, b1, W2, b2)


def _gru_msg(m0, m1, P, h, Wih, bih, Whh, bhh, W1, b1, W2, b2):
    return pl.pallas_call(
        _gru_msg_body,
        grid=(N // R,),
        in_specs=[
            _row_spec((R, H)), _row_spec((R, H)), _row_spec((R, H)),
            _row_spec((R, H)),
            _full_spec((3 * H, H)), _full_spec((1, 3 * H)),
            _full_spec((3 * H, H)), _full_spec((1, 3 * H)),
            _full_spec((H, H)), _full_spec((1, H)),
            _full_spec((H, H)), _full_spec((1, H)),
        ],
        out_specs=[_row_spec((R, H)), _row_spec((R, H))],
        out_shape=[jax.ShapeDtypeStruct((N, H), jnp.float32)] * 2,
    )(m0, m1, P, h, Wih, bih, Whh, bhh, W1, b1, W2, b2)


def _gru_last(m0, m1, P, h, Wih, bih, Whh, bhh):
    return pl.pallas_call(
        _gru_last_body,
        grid=(N // R,),
        in_specs=[
            _row_spec((R, H)), _row_spec((R, H)), _row_spec((R, H)),
            _row_spec((R, H)),
            _full_spec((3 * H, H)), _full_spec((1, 3 * H)),
            _full_spec((3 * H, H)), _full_spec((1, 3 * H)),
        ],
        out_specs=_row_spec((R, H)),
        out_shape=jax.ShapeDtypeStruct((N, H), jnp.float32),
    )(m0, m1, P, h, Wih, bih, Whh, bhh)


def _pool_head(h, batch2, W1, b1, W2, b2):
    return pl.pallas_call(
        _pool_head_body,
        grid=(1,),
        in_specs=[
            _full_spec((N, H)), _full_spec((N, 1)),
            _full_spec((H // 2, H)), _full_spec((1, H // 2)),
            _full_spec((1, H // 2)), _full_spec((1, 1)),
        ],
        out_specs=_full_spec((G, 1)),
        out_shape=jax.ShapeDtypeStruct((G, 1), jnp.float32),
    )(h, batch2, W1, b1, W2, b2)


def _make_scatter():
    mesh = plsc.VectorSubcoreMesh(core_axis_name="c", subcore_axis_name="s")

    @functools.partial(
        pl.kernel,
        out_type=jax.ShapeDtypeStruct((2, N, H), jnp.float32),
        mesh=mesh,
        scratch_types=[
            pltpu.VMEM_SHARED((N_ACC, H), jnp.float32),
            pltpu.VMEM((NCH, K), jnp.int32),
            pltpu.VMEM((NCH, K), jnp.int32),
            pltpu.VMEM((K, H), jnp.float32),
            pltpu.SemaphoreType.DMA,
        ],
    )
    def scatter_kernel(P_hbm, src_hbm, dst_hbm, m_hbm, acc, src_v, dst_v,
                       rows_v, sem):
        cid = lax.axis_index("c")
        sid = lax.axis_index("s")
        wid = sid * 2 + cid
        r0 = pl.multiple_of(sid * RPT, 8)
        pltpu.sync_copy(P_hbm.at[pl.ds(r0, RPT)], acc.at[pl.ds(r0, RPT)])

        @pl.when(sid == 15)
        def _():
            pltpu.sync_copy(P_hbm.at[pl.ds(16 * RPT, N - 16 * RPT)],
                            acc.at[pl.ds(16 * RPT, N - 16 * RPT)])

        pltpu.sync_copy(src_hbm.at[wid], src_v)
        pltpu.sync_copy(dst_hbm.at[wid], dst_v)
        plsc.subcore_barrier()

        def body(j, carry):
            pltpu.async_copy(P_hbm.at[src_v.at[j]], rows_v, sem).wait()
            pltpu.sync_copy(rows_v, acc.at[dst_v.at[j]], add=True)
            return carry

        lax.fori_loop(0, NCH, body, 0)
        plsc.subcore_barrier()
        pltpu.sync_copy(acc.at[pl.ds(r0, RPT)], m_hbm.at[cid, pl.ds(r0, RPT)])

        @pl.when(sid == 15)
        def _():
            pltpu.sync_copy(acc.at[pl.ds(16 * RPT, N - 16 * RPT)],
                            m_hbm.at[cid, pl.ds(16 * RPT, N - 16 * RPT)])

    return scatter_kernel


@functools.lru_cache(maxsize=None)
def _get_scatter():
    return _make_scatter()


def _scatter_edges(P, src_p, dst_p):
    return _get_scatter()(P, src_p, dst_p)


def kernel(x, edge_index, batch, enc_W, enc_b, msg_W1, msg_b1, msg_W2, msg_b2,
           gru_Wih, gru_bih, gru_Whh, gru_bhh, head_W1, head_b1, head_W2,
           head_b2):
    pad = E_PAD - E
    src_p = jnp.concatenate(
        [edge_index[0], jnp.zeros((pad,), jnp.int32)]).reshape(NW, NCH, K)
    dst_p = jnp.concatenate(
        [edge_index[1], jnp.full((pad,), N, jnp.int32)]).reshape(NW, NCH, K)

    h, P = _enc_msg(x, enc_W, enc_b.reshape(1, H),
                    msg_W1[0], msg_b1[0].reshape(1, H),
                    msg_W2[0], msg_b2[0].reshape(1, H))
    for l in range(L):
        m = _scatter_edges(P, src_p, dst_p)
        gru_args = (m[0], m[1], P, h,
                    gru_Wih[l], gru_bih[l].reshape(1, 3 * H),
                    gru_Whh[l], gru_bhh[l].reshape(1, 3 * H))
        if l < L - 1:
            h, P = _gru_msg(*gru_args,
                            msg_W1[l + 1], msg_b1[l + 1].reshape(1, H),
                            msg_W2[l + 1], msg_b2[l + 1].reshape(1, H))
        else:
            h = _gru_last(*gru_args)

    out = _pool_head(h, batch.reshape(N, 1),
                     head_W1, head_b1.reshape(1, H // 2),
                     head_W2, head_b2.reshape(1, 1))
    return out.reshape(G)
